# baseline (speedup 1.0000x reference)
import functools

import jax
import jax.numpy as jnp
from jax.experimental import pallas as pl
from jax.experimental.pallas import tpu as pltpu
from jax.experimental.pallas import tpu_sc as plsc

_E = 8
_D = 1024
_H = 2048
_N = 2048
_A = _N * 2
_T = 256
_MAX_TILES = 24
_NPAD = _MAX_TILES * _T


def _router(scores):
    def body(s_ref, o_ref):
        s = s_ref[...]
        i1 = jnp.argmax(s, axis=1).astype(jnp.int32)
        cols = jax.lax.broadcasted_iota(jnp.int32, s.shape, 1)
        s2 = jnp.where(cols == i1[:, None], -jnp.inf, s)
        i2 = jnp.argmax(s2, axis=1).astype(jnp.int32)
        o_ref[0, :] = i1
        o_ref[1, :] = i2

    out = pl.pallas_call(
        body,
        out_shape=jax.ShapeDtypeStruct((2, _N), jnp.int32),
    )(scores)
    return out


_SC_UNITS = 32


def _sc_gather(data, idx):
    n = idx.shape[0]
    d = data.shape[1]
    sub = d // 128
    n8 = n * sub
    per = n8 // _SC_UNITS
    assert per * _SC_UNITS == n8
    data128 = data.reshape(data.shape[0] * sub, 128)
    idx8 = (idx[:, None] * sub + jnp.arange(sub, dtype=jnp.int32)[None, :]
            ).reshape(1, n8)
    mesh = plsc.VectorSubcoreMesh(core_axis_name="core", subcore_axis_name="subcore")

    @functools.partial(
        pl.kernel,
        out_type=jax.ShapeDtypeStruct((n8, 128), data.dtype),
        mesh=mesh,
        scratch_types=[
            pltpu.VMEM((per,), jnp.int32),
            pltpu.VMEM((per, 128), data.dtype),
        ],
    )
    def k(x_hbm, i_hbm, o_hbm, ibuf, buf):
        c = jax.lax.axis_index("core")
        s = jax.lax.axis_index("subcore")
        base = (c * 16 + s) * per
        pltpu.sync_copy(i_hbm.at[0, pl.ds(base, per)], ibuf)
        pltpu.sync_copy(x_hbm.at[ibuf], buf)
        pltpu.sync_copy(buf, o_hbm.at[pl.ds(base, per)])

    return k(data128, idx8).reshape(n, d)


def _pack_bf16(a):
    n, d = a.shape
    return jax.lax.bitcast_convert_type(a.reshape(n, d // 2, 2), jnp.int32)


def _unpack_bf16(a):
    n, h = a.shape
    return jax.lax.bitcast_convert_type(a, jnp.bfloat16).reshape(n, 2 * h)


def _gelu_exact(x):
    return 0.5 * x * (1.0 + jax.lax.erf(x * 0.7071067811865476))


def _grouped_mlp(tile_expert, tile_used, xg, W1, b1, W2, b2, W3, b3):
    def body(te_ref, tu_ref, x_ref, w1_ref, b1_ref, w2_ref, b2_ref, w3_ref,
             b3_ref, o_ref):
        @pl.when(tu_ref[pl.program_id(0)] == 1)
        def _():
            xb = x_ref[...]
            h = jnp.dot(xb, w1_ref[0], preferred_element_type=jnp.float32)
            h = h + b1_ref[0]
            h = _gelu_exact(h).astype(jnp.bfloat16)
            h = jnp.dot(h, w2_ref[0], preferred_element_type=jnp.float32)
            h = h + b2_ref[0]
            h = _gelu_exact(h).astype(jnp.bfloat16)
            y = jnp.dot(h, w3_ref[0], preferred_element_type=jnp.float32)
            o_ref[...] = (y + b3_ref[0]).astype(jnp.bfloat16)

    grid_spec = pltpu.PrefetchScalarGridSpec(
        num_scalar_prefetch=2,
        grid=(_MAX_TILES,),
        in_specs=[
            pl.BlockSpec((_T, _D), lambda i, te, tu: (i, 0)),
            pl.BlockSpec((1, _D, _H), lambda i, te, tu: (te[i], 0, 0)),
            pl.BlockSpec((1, 1, _H), lambda i, te, tu: (te[i], 0, 0)),
            pl.BlockSpec((1, _H, _H), lambda i, te, tu: (te[i], 0, 0)),
            pl.BlockSpec((1, 1, _H), lambda i, te, tu: (te[i], 0, 0)),
            pl.BlockSpec((1, _H, _D), lambda i, te, tu: (te[i], 0, 0)),
            pl.BlockSpec((1, 1, _D), lambda i, te, tu: (te[i], 0, 0)),
        ],
        out_specs=pl.BlockSpec((_T, _D), lambda i, te, tu: (i, 0)),
    )
    return pl.pallas_call(
        body,
        grid_spec=grid_spec,
        out_shape=jax.ShapeDtypeStruct((_NPAD, _D), jnp.bfloat16),
    )(tile_expert, tile_used, xg, W1, b1, W2, b2, W3, b3)


def _combine(yc):
    def body(a_ref, o_ref):
        a = a_ref[...].astype(jnp.float32)
        o_ref[...] = 0.5 * (a[:, :_D] + a[:, _D:])

    yc2 = yc.reshape(_N, 2 * _D)
    return pl.pallas_call(
        body,
        grid=(8,),
        in_specs=[pl.BlockSpec((_N // 8, 2 * _D), lambda i: (i, 0))],
        out_specs=pl.BlockSpec((_N // 8, _D), lambda i: (i, 0)),
        out_shape=jax.ShapeDtypeStruct((_N, _D), jnp.float32),
    )(yc2)


def kernel(x, expert_tokens, W1, b1, W2, b2, W3, b3):
    x_flat = x.reshape(_N, _D)

    scores = jnp.matmul(x_flat, expert_tokens.T)

    top2 = _router(scores)
    e_flat = top2.T.reshape(_A)

    onehot = (e_flat[:, None] == jnp.arange(_E, dtype=jnp.int32)[None, :])
    onehot = onehot.astype(jnp.int32)
    counts = onehot.sum(axis=0)
    rank_all = jnp.cumsum(onehot, axis=0) - onehot
    rank = jnp.take_along_axis(rank_all, e_flat[:, None], axis=1)[:, 0]
    pad_counts = ((counts + _T - 1) // _T) * _T
    pad_ends = jnp.cumsum(pad_counts)
    pad_starts = pad_ends - pad_counts
    pos = pad_starts[e_flat] + rank
    row_tokens = jnp.zeros((_NPAD,), jnp.int32).at[pos].set(
        jnp.arange(_A, dtype=jnp.int32) // 2)
    total_pad = pad_ends[-1]
    tile_rows = jnp.arange(_MAX_TILES, dtype=jnp.int32) * _T
    tile_used = (tile_rows < total_pad).astype(jnp.int32)
    te_raw = jnp.searchsorted(pad_ends, tile_rows, side="right").astype(jnp.int32)
    te_tail = jnp.searchsorted(pad_ends, total_pad - 1, side="right").astype(jnp.int32)
    tile_expert = jnp.where(tile_used == 1, jnp.minimum(te_raw, _E - 1), te_tail)

    xg_packed = _sc_gather(_pack_bf16(x_flat.astype(jnp.bfloat16)), row_tokens)
    xg = _unpack_bf16(xg_packed)

    yg = _grouped_mlp(tile_expert, tile_used, xg,
                      W1.astype(jnp.bfloat16), b1.reshape(_E, 1, _H),
                      W2.astype(jnp.bfloat16), b2.reshape(_E, 1, _H),
                      W3.astype(jnp.bfloat16), b3.reshape(_E, 1, _D))

    yc = _unpack_bf16(_sc_gather(_pack_bf16(yg), pos))
    out = _combine(yc)
    return out.reshape(1, _N, _D)


# speedup vs baseline: 5.8855x; 5.8855x over previous
import functools

import jax
import jax.numpy as jnp
from jax.experimental import pallas as pl
from jax.experimental.pallas import tpu as pltpu
from jax.experimental.pallas import tpu_sc as plsc

_E = 8
_D = 1024
_H = 2048
_N = 2048
_A = _N * 2
_T = 256
_MAX_TILES = 24
_NPAD = _MAX_TILES * _T


def _router(scores):
    def body(s_ref, o_ref):
        s = s_ref[...]
        i1 = jnp.argmax(s, axis=1).astype(jnp.int32)
        cols = jax.lax.broadcasted_iota(jnp.int32, s.shape, 1)
        s2 = jnp.where(cols == i1[:, None], -jnp.inf, s)
        i2 = jnp.argmax(s2, axis=1).astype(jnp.int32)
        o_ref[0, :] = i1
        o_ref[1, :] = i2

    out = pl.pallas_call(
        body,
        out_shape=jax.ShapeDtypeStruct((2, _N), jnp.int32),
    )(scores)
    return out


_SC_UNITS = 32


def _sc_gather(data, idx):
    n = idx.shape[0]
    d = data.shape[1]
    sub = d // 128
    n8 = n * sub
    per = n8 // _SC_UNITS
    assert per * _SC_UNITS == n8
    data128 = data.reshape(data.shape[0] * sub, 128)
    idx8 = (idx[:, None] * sub + jnp.arange(sub, dtype=jnp.int32)[None, :]
            ).reshape(1, n8)
    mesh = plsc.VectorSubcoreMesh(core_axis_name="core", subcore_axis_name="subcore")

    @functools.partial(
        pl.kernel,
        out_type=jax.ShapeDtypeStruct((n8, 128), data.dtype),
        mesh=mesh,
        scratch_types=[
            pltpu.VMEM((per,), jnp.int32),
            pltpu.VMEM((per, 128), data.dtype),
        ],
    )
    def k(x_hbm, i_hbm, o_hbm, ibuf, buf):
        c = jax.lax.axis_index("core")
        s = jax.lax.axis_index("subcore")
        base = (c * 16 + s) * per
        pltpu.sync_copy(i_hbm.at[0, pl.ds(base, per)], ibuf)
        pltpu.sync_copy(x_hbm.at[ibuf], buf)
        pltpu.sync_copy(buf, o_hbm.at[pl.ds(base, per)])

    return k(data128, idx8).reshape(n, d)


_MASK_HI = -65536


def _pack_halves(y):
    k = y.shape[1] // 2
    a = y[:, :k].astype(jnp.bfloat16).astype(jnp.float32)
    b = y[:, k:].astype(jnp.bfloat16).astype(jnp.float32)
    ai = jax.lax.bitcast_convert_type(a, jnp.int32)
    bi = jax.lax.bitcast_convert_type(b, jnp.int32)
    return jax.lax.shift_right_logical(ai, 16) | (bi & _MASK_HI)


def _unpack_halves(p):
    lo = jax.lax.bitcast_convert_type(jax.lax.shift_left(p, 16), jnp.float32)
    hi = jax.lax.bitcast_convert_type(p & _MASK_HI, jnp.float32)
    return jnp.concatenate([lo, hi], axis=1)


def _pack_x(x_flat):
    def body(x_ref, o_ref):
        o_ref[...] = _pack_halves(x_ref[...])

    return pl.pallas_call(
        body,
        grid=(8,),
        in_specs=[pl.BlockSpec((_N // 8, _D), lambda i: (i, 0))],
        out_specs=pl.BlockSpec((_N // 8, _D // 2), lambda i: (i, 0)),
        out_shape=jax.ShapeDtypeStruct((_N, _D // 2), jnp.int32),
    )(x_flat)


def _gelu_exact(x):
    return 0.5 * x * (1.0 + jax.lax.erf(x * 0.7071067811865476))


def _grouped_mlp(tile_expert, tile_used, xg, W1, b1, W2, b2, W3, b3):
    def body(te_ref, tu_ref, x_ref, w1_ref, b1_ref, w2_ref, b2_ref, w3_ref,
             b3_ref, o_ref):
        @pl.when(tu_ref[pl.program_id(0)] == 1)
        def _():
            xb = _unpack_halves(x_ref[...]).astype(jnp.bfloat16)
            h = jnp.dot(xb, w1_ref[0], preferred_element_type=jnp.float32)
            h = h + b1_ref[0]
            h = _gelu_exact(h).astype(jnp.bfloat16)
            h = jnp.dot(h, w2_ref[0], preferred_element_type=jnp.float32)
            h = h + b2_ref[0]
            h = _gelu_exact(h).astype(jnp.bfloat16)
            y = jnp.dot(h, w3_ref[0], preferred_element_type=jnp.float32)
            o_ref[...] = _pack_halves(y + b3_ref[0])

    grid_spec = pltpu.PrefetchScalarGridSpec(
        num_scalar_prefetch=2,
        grid=(_MAX_TILES,),
        in_specs=[
            pl.BlockSpec((_T, _D // 2), lambda i, te, tu: (i, 0)),
            pl.BlockSpec((1, _D, _H), lambda i, te, tu: (te[i], 0, 0)),
            pl.BlockSpec((1, 1, _H), lambda i, te, tu: (te[i], 0, 0)),
            pl.BlockSpec((1, _H, _H), lambda i, te, tu: (te[i], 0, 0)),
            pl.BlockSpec((1, 1, _H), lambda i, te, tu: (te[i], 0, 0)),
            pl.BlockSpec((1, _H, _D), lambda i, te, tu: (te[i], 0, 0)),
            pl.BlockSpec((1, 1, _D), lambda i, te, tu: (te[i], 0, 0)),
        ],
        out_specs=pl.BlockSpec((_T, _D // 2), lambda i, te, tu: (i, 0)),
    )
    return pl.pallas_call(
        body,
        grid_spec=grid_spec,
        out_shape=jax.ShapeDtypeStruct((_NPAD, _D // 2), jnp.int32),
    )(tile_expert, tile_used, xg, W1, b1, W2, b2, W3, b3)


def _combine(yc):
    def body(a_ref, o_ref):
        a = a_ref[...]
        y0 = _unpack_halves(a[:, :_D // 2])
        y1 = _unpack_halves(a[:, _D // 2:])
        o_ref[...] = 0.5 * (y0 + y1)

    yc2 = yc.reshape(_N, _D)
    return pl.pallas_call(
        body,
        grid=(8,),
        in_specs=[pl.BlockSpec((_N // 8, _D), lambda i: (i, 0))],
        out_specs=pl.BlockSpec((_N // 8, _D), lambda i: (i, 0)),
        out_shape=jax.ShapeDtypeStruct((_N, _D), jnp.float32),
    )(yc2)


def kernel(x, expert_tokens, W1, b1, W2, b2, W3, b3):
    x_flat = x.reshape(_N, _D)

    scores = jnp.matmul(x_flat, expert_tokens.T)

    top2 = _router(scores)
    e_flat = top2.T.reshape(_A)

    onehot = (e_flat[:, None] == jnp.arange(_E, dtype=jnp.int32)[None, :])
    onehot = onehot.astype(jnp.int32)
    counts = onehot.sum(axis=0)
    rank_all = jnp.cumsum(onehot, axis=0) - onehot
    rank = jnp.take_along_axis(rank_all, e_flat[:, None], axis=1)[:, 0]
    pad_counts = ((counts + _T - 1) // _T) * _T
    pad_ends = jnp.cumsum(pad_counts)
    pad_starts = pad_ends - pad_counts
    pos = pad_starts[e_flat] + rank
    row_tokens = jnp.zeros((_NPAD,), jnp.int32).at[pos].set(
        jnp.arange(_A, dtype=jnp.int32) // 2)
    total_pad = pad_ends[-1]
    tile_rows = jnp.arange(_MAX_TILES, dtype=jnp.int32) * _T
    tile_used = (tile_rows < total_pad).astype(jnp.int32)
    te_raw = jnp.searchsorted(pad_ends, tile_rows, side="right").astype(jnp.int32)
    te_tail = jnp.searchsorted(pad_ends, total_pad - 1, side="right").astype(jnp.int32)
    tile_expert = jnp.where(tile_used == 1, jnp.minimum(te_raw, _E - 1), te_tail)

    xg = _sc_gather(_pack_x(x_flat), row_tokens)

    yg = _grouped_mlp(tile_expert, tile_used, xg,
                      W1.astype(jnp.bfloat16), b1.reshape(_E, 1, _H),
                      W2.astype(jnp.bfloat16), b2.reshape(_E, 1, _H),
                      W3.astype(jnp.bfloat16), b3.reshape(_E, 1, _D))

    yc = _sc_gather(yg, pos)
    out = _combine(yc)
    return out.reshape(1, _N, _D)
